# baseline (speedup 1.0000x reference)
import functools

import jax
import jax.numpy as jnp
import numpy as np
from jax.experimental import pallas as pl
from jax.experimental.pallas import tpu as pltpu

_WIDTH = 64
_K = 8192
_TEMP = 0.5
_TN = 224


def _quant_block(scale_ref, z_ref, c_ref, zq_ref, pt_ref, lse_ref):
    @pl.when(pl.program_id(0) == 0)
    def _init():
        pt_ref[...] = jnp.zeros((1, 1), jnp.float32)
        lse_ref[...] = jnp.zeros((1, 1), jnp.float32)

    z = z_ref[...]
    c = c_ref[...]
    scale = scale_ref[0]

    zsq = jnp.sum(z * z, axis=1, keepdims=True)
    csq = jnp.sum(c * c, axis=1)[None, :]
    s = jax.lax.dot_general(z, c, (((1,), (1,)), ((), ())),
                            preferred_element_type=jnp.float32,
                            precision=jax.lax.Precision.HIGHEST)
    t = (zsq - 2.0 * s + csq) * scale
    m = jnp.max(t, axis=1, keepdims=True)
    e = jnp.exp(t - m)
    den = jnp.sum(e, axis=1, keepdims=True)
    p = e / den
    zq = jax.lax.dot_general(p, c, (((1,), (0,)), ((), ())),
                             preferred_element_type=jnp.float32,
                             precision=jax.lax.Precision.HIGHEST)
    zq_ref[...] = zq
    pt_ref[...] += jnp.sum(p * t).reshape(1, 1)
    lse_ref[...] += jnp.sum(jnp.log(den) + m).reshape(1, 1)


@functools.partial(jax.jit, static_argnames=())
def _quantize(zf, codebook, var):
    n = zf.shape[0]
    scale = (-0.5 / (var * _TEMP)).reshape(1).astype(jnp.float32)
    grid = (n // _TN,)
    zq, pt, lse = pl.pallas_call(
        _quant_block,
        grid=grid,
        in_specs=[
            pl.BlockSpec(memory_space=pltpu.SMEM),
            pl.BlockSpec((_TN, _WIDTH), lambda i: (i, 0)),
            pl.BlockSpec((_K, _WIDTH), lambda i: (0, 0)),
        ],
        out_specs=[
            pl.BlockSpec((_TN, _WIDTH), lambda i: (i, 0)),
            pl.BlockSpec((1, 1), lambda i: (0, 0)),
            pl.BlockSpec((1, 1), lambda i: (0, 0)),
        ],
        out_shape=[
            jax.ShapeDtypeStruct((n, _WIDTH), jnp.float32),
            jax.ShapeDtypeStruct((1, 1), jnp.float32),
            jax.ShapeDtypeStruct((1, 1), jnp.float32),
        ],
    )(scale, zf, codebook)
    return zq, pt[0, 0], lse[0, 0]


def _conv_s2(x, w, b):
    y = jax.lax.conv_general_dilated(x, w, window_strides=(2, 2), padding=((1, 1), (1, 1)),
                                     dimension_numbers=('NCHW', 'OIHW', 'NCHW'))
    return y + b[None, :, None, None]


def _tconv_s2(x, w, b):
    y = jax.lax.conv_general_dilated(x, w[:, :, ::-1, ::-1], window_strides=(1, 1),
                                     padding=((2, 2), (2, 2)), lhs_dilation=(2, 2),
                                     dimension_numbers=('NCHW', 'OIHW', 'NCHW'))
    return y + b[None, :, None, None]


def kernel(x, enc_w1, enc_b1, enc_w2, enc_b2, enc_w3, enc_b3,
           dec_w1, dec_b1, dec_w2, dec_b2, dec_w3, dec_b3, codebook, log_var):
    h = jax.nn.relu(_conv_s2(x, enc_w1, enc_b1))
    h = jax.nn.relu(_conv_s2(h, enc_w2, enc_b2))
    z = _conv_s2(h, enc_w3, enc_b3)
    B, Cz, Hh, Ww = z.shape
    zf = jnp.transpose(z, (0, 2, 3, 1)).reshape(-1, Cz)
    n = zf.shape[0]

    var = jnp.exp(log_var)
    zq, pt_sum, lse_sum = _quantize(zf, codebook, var)
    mean_pt = pt_sum / n
    mean_lse = lse_sum / n
    loss_latent = (1.0 - _TEMP) * mean_pt - mean_lse + np.float32(np.log(_K))

    z_q4 = jnp.transpose(zq.reshape(B, Hh, Ww, Cz), (0, 3, 1, 2))
    h = jax.nn.relu(_tconv_s2(z_q4, dec_w1, dec_b1))
    h = jax.nn.relu(_tconv_s2(h, dec_w2, dec_b2))
    x_rec = _tconv_s2(h, dec_w3, dec_b3)
    dim_x = float(np.prod(x_rec.shape[1:]))
    se = jnp.sum((x_rec - x) ** 2) / B
    loss_rec = dim_x * jnp.log(se) / 2.0
    rmse = jnp.sqrt(se / dim_x)
    loss = loss_latent + loss_rec
    return (loss, x_rec, rmse)


# speedup vs baseline: 1.6903x; 1.6903x over previous
import functools

import jax
import jax.numpy as jnp
import numpy as np
from jax.experimental import pallas as pl
from jax.experimental.pallas import tpu as pltpu

_WIDTH = 64
_K = 8192
_TEMP = 0.5
_TN = 224


def _quant_block(scale_ref, z_ref, caug_ref, csqr_ref, zq_ref, pt_ref, lse_ref):
    @pl.when(pl.program_id(0) == 0)
    def _init():
        pt_ref[...] = jnp.zeros((1, 1), jnp.float32)
        lse_ref[...] = jnp.zeros((1, 1), jnp.float32)

    z = z_ref[...]
    caug = caug_ref[...]
    csqr = csqr_ref[...]
    scale = scale_ref[0]

    zsq = jnp.sum(z * z, axis=1, keepdims=True)
    s = jax.lax.dot_general(z, caug[:, :_WIDTH], (((1,), (1,)), ((), ())),
                            preferred_element_type=jnp.float32)
    g = csqr - 2.0 * s
    mg = jnp.min(g, axis=1, keepdims=True)
    e = jnp.exp((g - mg) * scale)
    r = jax.lax.dot_general(e, caug, (((1,), (0,)), ((), ())),
                            preferred_element_type=jnp.float32)
    den = r[:, _WIDTH + 1:_WIDTH + 2]
    ecsq = r[:, _WIDTH:_WIDTH + 1]
    zq = r[:, :_WIDTH] / den
    zq_ref[...] = zq
    zdotzq = jnp.sum(z * zq, axis=1, keepdims=True)
    pt_row = scale * (zsq + ecsq / den - 2.0 * zdotzq)
    m = scale * (zsq + mg)
    lse_row = jnp.log(den) + m
    pt_ref[...] += jnp.sum(pt_row).reshape(1, 1)
    lse_ref[...] += jnp.sum(lse_row).reshape(1, 1)


@functools.partial(jax.jit, static_argnames=())
def _quantize(zf, codebook, var):
    n = zf.shape[0]
    k = codebook.shape[0]
    scale = (-0.5 / (var * _TEMP)).reshape(1).astype(jnp.float32)
    csq = jnp.sum(codebook * codebook, axis=1)
    caug = jnp.concatenate(
        [codebook, csq[:, None], jnp.ones((k, 1), jnp.float32),
         jnp.zeros((k, 128 - _WIDTH - 2), jnp.float32)], axis=1)
    csqr = csq[None, :]
    grid = (n // _TN,)
    zq, pt, lse = pl.pallas_call(
        _quant_block,
        grid=grid,
        in_specs=[
            pl.BlockSpec(memory_space=pltpu.SMEM),
            pl.BlockSpec((_TN, _WIDTH), lambda i: (i, 0)),
            pl.BlockSpec((_K, 128), lambda i: (0, 0)),
            pl.BlockSpec((1, _K), lambda i: (0, 0)),
        ],
        out_specs=[
            pl.BlockSpec((_TN, _WIDTH), lambda i: (i, 0)),
            pl.BlockSpec((1, 1), lambda i: (0, 0)),
            pl.BlockSpec((1, 1), lambda i: (0, 0)),
        ],
        out_shape=[
            jax.ShapeDtypeStruct((n, _WIDTH), jnp.float32),
            jax.ShapeDtypeStruct((1, 1), jnp.float32),
            jax.ShapeDtypeStruct((1, 1), jnp.float32),
        ],
    )(scale, zf, caug, csqr)
    return zq, pt[0, 0], lse[0, 0]


def _conv_s2(x, w, b):
    y = jax.lax.conv_general_dilated(x, w, window_strides=(2, 2), padding=((1, 1), (1, 1)),
                                     dimension_numbers=('NCHW', 'OIHW', 'NCHW'))
    return y + b[None, :, None, None]


def _tconv_s2(x, w, b):
    y = jax.lax.conv_general_dilated(x, w[:, :, ::-1, ::-1], window_strides=(1, 1),
                                     padding=((2, 2), (2, 2)), lhs_dilation=(2, 2),
                                     dimension_numbers=('NCHW', 'OIHW', 'NCHW'))
    return y + b[None, :, None, None]


def kernel(x, enc_w1, enc_b1, enc_w2, enc_b2, enc_w3, enc_b3,
           dec_w1, dec_b1, dec_w2, dec_b2, dec_w3, dec_b3, codebook, log_var):
    h = jax.nn.relu(_conv_s2(x, enc_w1, enc_b1))
    h = jax.nn.relu(_conv_s2(h, enc_w2, enc_b2))
    z = _conv_s2(h, enc_w3, enc_b3)
    B, Cz, Hh, Ww = z.shape
    zf = jnp.transpose(z, (0, 2, 3, 1)).reshape(-1, Cz)
    n = zf.shape[0]

    var = jnp.exp(log_var)
    zq, pt_sum, lse_sum = _quantize(zf, codebook, var)
    mean_pt = pt_sum / n
    mean_lse = lse_sum / n
    loss_latent = (1.0 - _TEMP) * mean_pt - mean_lse + np.float32(np.log(_K))

    z_q4 = jnp.transpose(zq.reshape(B, Hh, Ww, Cz), (0, 3, 1, 2))
    h = jax.nn.relu(_tconv_s2(z_q4, dec_w1, dec_b1))
    h = jax.nn.relu(_tconv_s2(h, dec_w2, dec_b2))
    x_rec = _tconv_s2(h, dec_w3, dec_b3)
    dim_x = float(np.prod(x_rec.shape[1:]))
    se = jnp.sum((x_rec - x) ** 2) / B
    loss_rec = dim_x * jnp.log(se) / 2.0
    rmse = jnp.sqrt(se / dim_x)
    loss = loss_latent + loss_rec
    return (loss, x_rec, rmse)
